# baseline (speedup 1.0000x reference)
import functools

import jax
import jax.numpy as jnp
from jax import lax
from jax.experimental import pallas as pl
from jax.experimental.pallas import tpu as pltpu
from jax.experimental.pallas import tpu_sc as plsc

D_MODEL = 1024
N_ROWS = 13
BATCH = 16384
NUM_SUBCORES = 16
GROUPS_C0 = 30
GROUPS_C1 = 34
ROWS_C0 = GROUPS_C0 * 16
ROWS_C1 = GROUPS_C1 * 16
assert (ROWS_C0 + ROWS_C1) * NUM_SUBCORES == BATCH

_mesh = plsc.VectorSubcoreMesh(core_axis_name="c", subcore_axis_name="s")


@functools.partial(
    pl.kernel,
    mesh=_mesh,
    out_type=jax.ShapeDtypeStruct((BATCH, D_MODEL), jnp.float32),
    compiler_params=pltpu.CompilerParams(use_tc_tiling_on_sc=False),
    scratch_types=[
        pltpu.VMEM((ROWS_C1,), jnp.int32),
        pltpu.VMEM((N_ROWS, D_MODEL), jnp.float32),
        pltpu.VMEM((16, D_MODEL), jnp.float32),
        pltpu.SemaphoreType.DMA,
    ],
)
def _emb_lookup(weeks_hbm, table_hbm, out_hbm, idx_v, table_v, drain_v, sem):
    c = lax.axis_index("c")
    s = lax.axis_index("s")
    is_c0 = c == 0
    base = jnp.where(is_c0, s * ROWS_C0, NUM_SUBCORES * ROWS_C0 + s * ROWS_C1)
    n_groups = jnp.where(is_c0, GROUPS_C0, GROUPS_C1)
    stage_t = pltpu.async_copy(table_hbm, table_v, sem)
    stage_i = pltpu.async_copy(weeks_hbm.at[pl.ds(base, ROWS_C1)], idx_v, sem)
    stage_t.wait()
    stage_i.wait()

    def issue(g, carry):
        v = idx_v[pl.ds(g * 16, 16)]
        for j in range(16):
            pltpu.async_copy(
                table_v.at[pl.ds(v[j], 1)],
                out_hbm.at[pl.ds(base + g * 16 + j, 1)],
                sem,
            )
        return carry

    lax.fori_loop(0, n_groups, issue, 0)

    def drain(g, carry):
        pltpu.make_async_copy(drain_v, out_hbm.at[pl.ds(base, 16)], sem).wait()
        return carry

    lax.fori_loop(0, n_groups, drain, 0)


def kernel(weeks, table):
    out = _emb_lookup(weeks.astype(jnp.int32), table)
    return out[:, None, :]


# speedup vs baseline: 1.0052x; 1.0052x over previous
import functools

import jax
import jax.numpy as jnp
from jax import lax
from jax.experimental import pallas as pl
from jax.experimental.pallas import tpu as pltpu
from jax.experimental.pallas import tpu_sc as plsc

D_MODEL = 1024
N_ROWS = 13
BATCH = 16384
NUM_SUBCORES = 16
GROUPS_C0 = 30
GROUPS_C1 = 34
ROWS_C0 = GROUPS_C0 * 16
ROWS_C1 = GROUPS_C1 * 16
assert (ROWS_C0 + ROWS_C1) * NUM_SUBCORES == BATCH

_mesh = plsc.VectorSubcoreMesh(core_axis_name="c", subcore_axis_name="s")


@functools.partial(
    pl.kernel,
    mesh=_mesh,
    out_type=jax.ShapeDtypeStruct((BATCH, D_MODEL), jnp.float32),
    compiler_params=pltpu.CompilerParams(use_tc_tiling_on_sc=False),
    scratch_types=[
        pltpu.VMEM((ROWS_C1,), jnp.int32),
        pltpu.VMEM((N_ROWS, D_MODEL), jnp.float32),
        pltpu.VMEM((16, D_MODEL), jnp.float32),
        pltpu.SemaphoreType.DMA,
    ],
)
def _emb_lookup(weeks_hbm, table_hbm, out_hbm, idx_v, table_v, drain_v, sem):
    c = lax.axis_index("c")
    s = lax.axis_index("s")
    is_c0 = c == 0
    base = jnp.where(is_c0, s * ROWS_C0, NUM_SUBCORES * ROWS_C0 + s * ROWS_C1)
    n_groups = jnp.where(is_c0, GROUPS_C0, GROUPS_C1)
    pltpu.sync_copy(table_hbm, table_v)
    pltpu.sync_copy(weeks_hbm.at[pl.ds(base, ROWS_C1)], idx_v)

    def issue(g, carry):
        v = idx_v[pl.ds(g * 16, 16)]
        for j in range(16):
            pltpu.async_copy(
                table_v.at[pl.ds(v[j], 1)],
                out_hbm.at[pl.ds(base + g * 16 + j, 1)],
                sem,
            )
        return carry

    lax.fori_loop(0, n_groups, issue, 0)

    def drain(g, carry):
        pltpu.make_async_copy(drain_v, out_hbm.at[pl.ds(base, 16)], sem).wait()
        return carry

    lax.fori_loop(0, n_groups, drain, 0)


def kernel(weeks, table):
    out = _emb_lookup(weeks.astype(jnp.int32), table)
    return out[:, None, :]
